# baseline (speedup 1.0000x reference)
import functools

import jax
import jax.numpy as jnp
from jax import lax
from jax.experimental import pallas as pl
from jax.experimental.pallas import tpu as pltpu
from jax.experimental.pallas import tpu_sc as plsc

L = 16
NW = 32
NB = 6
LEAD = 3


def _sc_kernel(b, s, d, n_tab, chunk_rows):
    rows_per_worker = (b * s) // NW
    workers_per_batch = s // rows_per_worker
    n_chunks = rows_per_worker // chunk_rows
    n_strips = d // L
    mesh = plsc.VectorSubcoreMesh(core_axis_name="c", subcore_axis_name="s")

    @functools.partial(
        pl.kernel,
        mesh=mesh,
        out_type=jax.ShapeDtypeStruct((b, s, d), jnp.float32),
        scratch_types=[
            pltpu.VMEM((n_tab, d), jnp.float32),
            pltpu.VMEM((rows_per_worker + L,), jnp.int32),
            pltpu.VMEM((NB, chunk_rows, d), jnp.float32),
            pltpu.SemaphoreType.DMA,
            pltpu.SemaphoreType.DMA,
            pltpu.SemaphoreType.DMA,
            pltpu.SemaphoreType.DMA,
            pltpu.SemaphoreType.DMA,
            pltpu.SemaphoreType.DMA,
            pltpu.SemaphoreType.DMA,
            pltpu.SemaphoreType.DMA,
            pltpu.SemaphoreType.DMA,
            pltpu.SemaphoreType.DMA,
            pltpu.SemaphoreType.DMA,
            pltpu.SemaphoreType.DMA,
            pltpu.SemaphoreType.DMA,
        ],
    )
    def k(x_hbm, ids_hbm, tab_hbm, out_hbm, tab_v, ids_v, buf,
          sem_t, in0, in1, in2, in3, in4, in5, out0, out1, out2, out3,
          out4, out5):
        wid = lax.axis_index("s") * 2 + lax.axis_index("c")
        bw = wid // workers_per_batch
        row0 = (wid % workers_per_batch) * rows_per_worker
        in_sems = (in0, in1, in2, in3, in4, in5)
        out_sems = (out0, out1, out2, out3, out4, out5)

        def start_in(c, bi):
            pltpu.make_async_copy(
                x_hbm.at[bw, pl.ds(row0 + c * chunk_rows, chunk_rows), :],
                buf.at[bi], in_sems[bi]).start()

        def wait_in(bi):
            pltpu.make_async_copy(
                x_hbm.at[bw, pl.ds(0, chunk_rows), :], buf.at[bi],
                in_sems[bi]).wait()

        def start_out(c, bi):
            pltpu.make_async_copy(
                buf.at[bi],
                out_hbm.at[bw, pl.ds(row0 + c * chunk_rows, chunk_rows), :],
                out_sems[bi]).start()

        def wait_out(bi):
            pltpu.make_async_copy(
                buf.at[bi], out_hbm.at[bw, pl.ds(0, chunk_rows), :],
                out_sems[bi]).wait()

        def compute(c, bi):
            idv = ids_v[pl.ds(c * chunk_rows, L)]
            k0 = idv[0]
            uniform = k0 == idv[chunk_rows - 1]

            @pl.when(uniform)
            def _fast():
                @plsc.parallel_loop(0, n_strips, unroll=4)
                def _strips(j):
                    o = j * L
                    tv = tab_v[k0, pl.ds(o, L)]
                    for r in range(chunk_rows):
                        plsc.addupdate(buf.at[bi, r, pl.ds(o, L)], tv)

            @pl.when(jnp.logical_not(uniform))
            def _slow():
                for r in range(chunk_rows):
                    kr = idv[r]

                    @plsc.parallel_loop(0, n_strips, unroll=4)
                    def _strips(j):
                        o = j * L
                        plsc.addupdate(buf.at[bi, r, pl.ds(o, L)],
                                       tab_v[kr, pl.ds(o, L)])

        for c in range(NB):
            start_in(c, c)
        tab_copy = pltpu.make_async_copy(tab_hbm, tab_v, sem_t)
        tab_copy.start()
        ids_copy = pltpu.make_async_copy(
            ids_hbm.at[bw, pl.ds(row0, rows_per_worker)],
            ids_v.at[pl.ds(0, rows_per_worker)], sem_t)
        ids_copy.start()
        tab_copy.wait()
        ids_copy.wait()

        def group(g, _):
            for u in range(NB):
                c = NB * g + u
                bi = u
                wait_in(bi)

                @pl.when(c >= NB - LEAD)
                def _():
                    wait_out((u + LEAD) % NB)
                    start_in(c + LEAD, (u + LEAD) % NB)

                compute(c, bi)
                start_out(c, bi)
            return 0

        n_groups = (n_chunks - LEAD) // NB
        lax.fori_loop(0, n_groups, group, 0)
        for c in range(n_groups * NB, n_chunks):
            bi = c % NB
            wait_in(bi)
            if c + LEAD < n_chunks:
                wait_out((bi + LEAD) % NB)
                start_in(c + LEAD, (bi + LEAD) % NB)
            compute(c, bi)
            start_out(c, bi)
        for bi in range(NB):
            wait_out(bi)

    return k


def kernel(x, segment_ids, segment_embedding):
    b, s, d = x.shape
    n_tab = segment_embedding.shape[0]
    out = _sc_kernel(b, s, d, n_tab, 8)(
        x, segment_ids.astype(jnp.int32), segment_embedding)
    return out


# speedup vs baseline: 1.0201x; 1.0201x over previous
import functools

import jax
import jax.numpy as jnp
from jax import lax
from jax.experimental import pallas as pl
from jax.experimental.pallas import tpu as pltpu
from jax.experimental.pallas import tpu_sc as plsc

L = 16
NW = 32
NBI = 4
NBO = 2


def _sc_kernel(b, s, d, n_tab, chunk_rows):
    rows_per_worker = (b * s) // NW
    workers_per_batch = s // rows_per_worker
    n_chunks = rows_per_worker // chunk_rows
    n_strips = d // L
    mesh = plsc.VectorSubcoreMesh(core_axis_name="c", subcore_axis_name="s")

    @functools.partial(
        pl.kernel,
        mesh=mesh,
        out_type=jax.ShapeDtypeStruct((b, s, d), jnp.float32),
        scratch_types=[
            pltpu.VMEM((n_tab, d), jnp.float32),
            pltpu.VMEM((rows_per_worker + L,), jnp.int32),
            pltpu.VMEM((NBI, chunk_rows, d), jnp.float32),
            pltpu.VMEM((NBO, chunk_rows, d), jnp.float32),
            pltpu.SemaphoreType.DMA,
            pltpu.SemaphoreType.DMA,
            pltpu.SemaphoreType.DMA,
            pltpu.SemaphoreType.DMA,
            pltpu.SemaphoreType.DMA,
            pltpu.SemaphoreType.DMA,
            pltpu.SemaphoreType.DMA,
        ],
    )
    def k(x_hbm, ids_hbm, tab_hbm, out_hbm, tab_v, ids_v, inb, outb,
          sem_t, in0, in1, in2, in3, out0, out1):
        wid = lax.axis_index("s") * 2 + lax.axis_index("c")
        bw = wid // workers_per_batch
        row0 = (wid % workers_per_batch) * rows_per_worker
        in_sems = (in0, in1, in2, in3)
        out_sems = (out0, out1)


        def start_in(c, bi):
            pltpu.make_async_copy(
                x_hbm.at[bw, pl.ds(row0 + c * chunk_rows, chunk_rows), :],
                inb.at[bi], in_sems[bi]).start()

        def wait_in(bi):
            pltpu.make_async_copy(
                x_hbm.at[bw, pl.ds(0, chunk_rows), :], inb.at[bi],
                in_sems[bi]).wait()

        def start_out(c, bo):
            pltpu.make_async_copy(
                outb.at[bo],
                out_hbm.at[bw, pl.ds(row0 + c * chunk_rows, chunk_rows), :],
                out_sems[bo]).start()

        def wait_out(bo):
            pltpu.make_async_copy(
                outb.at[bo], out_hbm.at[bw, pl.ds(0, chunk_rows), :],
                out_sems[bo]).wait()

        def compute(c, bi, bo):
            idv = ids_v[pl.ds(c * chunk_rows, L)]
            k0 = idv[0]
            uniform = k0 == idv[chunk_rows - 1]

            @pl.when(uniform)
            def _fast():
                @plsc.parallel_loop(0, n_strips, unroll=4)
                def _strips(j):
                    o = j * L
                    tv = tab_v[k0, pl.ds(o, L)]
                    for r in range(chunk_rows):
                        outb[bo, r, pl.ds(o, L)] = inb[bi, r, pl.ds(o, L)] + tv

            @pl.when(jnp.logical_not(uniform))
            def _slow():
                for r in range(chunk_rows):
                    kr = idv[r]

                    @plsc.parallel_loop(0, n_strips, unroll=4)
                    def _strips(j):
                        o = j * L
                        outb[bo, r, pl.ds(o, L)] = (
                            inb[bi, r, pl.ds(o, L)] + tab_v[kr, pl.ds(o, L)])

        for c in range(NBI):
            start_in(c, c)
        tab_copy = pltpu.make_async_copy(tab_hbm, tab_v, sem_t)
        tab_copy.start()
        ids_copy = pltpu.make_async_copy(
            ids_hbm.at[bw, pl.ds(row0, rows_per_worker)],
            ids_v.at[pl.ds(0, rows_per_worker)], sem_t)
        ids_copy.start()
        tab_copy.wait()
        ids_copy.wait()

        def group(g, _):
            for u in range(NBI):
                c = NBI * g + u
                bi = u
                bo = u % NBO
                wait_in(bi)

                @pl.when(c >= NBO)
                def _():
                    wait_out(bo)

                compute(c, bi, bo)
                start_out(c, bo)

                @pl.when(c + NBI < n_chunks)
                def _():
                    start_in(c + NBI, bi)
            return 0

        n_groups = n_chunks // NBI
        lax.fori_loop(0, n_groups, group, 0)
        for c in range(n_groups * NBI, n_chunks):
            bi = c % NBI
            bo = c % NBO
            wait_in(bi)
            wait_out(bo)
            compute(c, bi, bo)
            start_out(c, bo)
        for bo in range(NBO):
            wait_out(bo)

    return k


def kernel(x, segment_ids, segment_embedding):
    b, s, d = x.shape
    n_tab = segment_embedding.shape[0]
    out = _sc_kernel(b, s, d, n_tab, 8)(
        x, segment_ids.astype(jnp.int32), segment_embedding)
    return out


# speedup vs baseline: 1.0271x; 1.0069x over previous
import functools

import jax
import jax.numpy as jnp
from jax import lax
from jax.experimental import pallas as pl
from jax.experimental.pallas import tpu as pltpu
from jax.experimental.pallas import tpu_sc as plsc

L = 16
NW = 32
NBI = 4
NBO = 2


def _sc_kernel(b, s, d, n_tab, chunk_rows):
    rows_per_worker = (b * s) // NW
    workers_per_batch = s // rows_per_worker
    n_chunks = rows_per_worker // chunk_rows
    n_strips = d // L
    mesh = plsc.VectorSubcoreMesh(core_axis_name="c", subcore_axis_name="s")

    @functools.partial(
        pl.kernel,
        mesh=mesh,
        out_type=jax.ShapeDtypeStruct((b, s, d), jnp.float32),
        scratch_types=[
            pltpu.VMEM((n_tab, d), jnp.float32),
            pltpu.VMEM((rows_per_worker + L,), jnp.int32),
            pltpu.VMEM((NBI, chunk_rows, d), jnp.float32),
            pltpu.VMEM((NBO, chunk_rows, d), jnp.float32),
            pltpu.SemaphoreType.DMA,
            pltpu.SemaphoreType.DMA,
            pltpu.SemaphoreType.DMA,
            pltpu.SemaphoreType.DMA,
            pltpu.SemaphoreType.DMA,
            pltpu.SemaphoreType.DMA,
            pltpu.SemaphoreType.DMA,
        ],
    )
    def k(x_hbm, ids_hbm, tab_hbm, out_hbm, tab_v, ids_v, inb, outb,
          sem_t, in0, in1, in2, in3, out0, out1):
        wid = lax.axis_index("c") * 16 + lax.axis_index("s")
        bw = wid // workers_per_batch
        row0 = (wid % workers_per_batch) * rows_per_worker
        in_sems = (in0, in1, in2, in3)
        out_sems = (out0, out1)


        def start_in(c, bi):
            pltpu.make_async_copy(
                x_hbm.at[bw, pl.ds(row0 + c * chunk_rows, chunk_rows), :],
                inb.at[bi], in_sems[bi]).start()

        def wait_in(bi):
            pltpu.make_async_copy(
                x_hbm.at[bw, pl.ds(0, chunk_rows), :], inb.at[bi],
                in_sems[bi]).wait()

        def start_out(c, bo):
            pltpu.make_async_copy(
                outb.at[bo],
                out_hbm.at[bw, pl.ds(row0 + c * chunk_rows, chunk_rows), :],
                out_sems[bo]).start()

        def wait_out(bo):
            pltpu.make_async_copy(
                outb.at[bo], out_hbm.at[bw, pl.ds(0, chunk_rows), :],
                out_sems[bo]).wait()

        def compute(c, bi, bo):
            idv = ids_v[pl.ds(c * chunk_rows, L)]
            k0 = idv[0]
            uniform = k0 == idv[chunk_rows - 1]

            @pl.when(uniform)
            def _fast():
                @plsc.parallel_loop(0, n_strips, unroll=4)
                def _strips(j):
                    o = j * L
                    tv = tab_v[k0, pl.ds(o, L)]
                    for r in range(chunk_rows):
                        outb[bo, r, pl.ds(o, L)] = inb[bi, r, pl.ds(o, L)] + tv

            @pl.when(jnp.logical_not(uniform))
            def _slow():
                for r in range(chunk_rows):
                    kr = idv[r]

                    @plsc.parallel_loop(0, n_strips, unroll=4)
                    def _strips(j):
                        o = j * L
                        outb[bo, r, pl.ds(o, L)] = (
                            inb[bi, r, pl.ds(o, L)] + tab_v[kr, pl.ds(o, L)])

        for c in range(NBI):
            start_in(c, c)
        tab_copy = pltpu.make_async_copy(tab_hbm, tab_v, sem_t)
        tab_copy.start()
        ids_copy = pltpu.make_async_copy(
            ids_hbm.at[bw, pl.ds(row0, rows_per_worker)],
            ids_v.at[pl.ds(0, rows_per_worker)], sem_t)
        ids_copy.start()
        tab_copy.wait()
        ids_copy.wait()

        def group(g, _):
            for u in range(NBI):
                c = NBI * g + u
                bi = u
                bo = u % NBO
                wait_in(bi)

                @pl.when(c >= NBO)
                def _():
                    wait_out(bo)

                compute(c, bi, bo)
                start_out(c, bo)

                @pl.when(c + NBI < n_chunks)
                def _():
                    start_in(c + NBI, bi)
            return 0

        n_groups = n_chunks // NBI
        lax.fori_loop(0, n_groups, group, 0)
        for c in range(n_groups * NBI, n_chunks):
            bi = c % NBI
            bo = c % NBO
            wait_in(bi)
            wait_out(bo)
            compute(c, bi, bo)
            start_out(c, bo)
        for bo in range(NBO):
            wait_out(bo)

    return k


def kernel(x, segment_ids, segment_embedding):
    b, s, d = x.shape
    n_tab = segment_embedding.shape[0]
    out = _sc_kernel(b, s, d, n_tab, 8)(
        x, segment_ids.astype(jnp.int32), segment_embedding)
    return out
